# baseline (speedup 1.0000x reference)
import functools
import math

import jax
import jax.numpy as jnp
from jax import lax
from jax.experimental import pallas as pl
from jax.experimental.pallas import tpu as pltpu
from jax.experimental.pallas import tpu_sc as plsc

HID = 128
NHEADS = 8
ATOM_CUTOFF = 5.0
GRID_CUTOFF = 8.0
LOG2 = math.log(2.0)

NC = 2
NS = 16
NW = NC * NS
KCH = 128


def _ssp(x):
    return jax.nn.softplus(x) - LOG2


def _ln(x, g, b):
    m = jnp.mean(x, axis=-1, keepdims=True)
    v = jnp.var(x, axis=-1, keepdims=True)
    return (x - m) * jax.lax.rsqrt(v + 1e-5) * g + b


def _full(shape):
    return pl.BlockSpec(shape, lambda *_: tuple(0 for _ in shape))


def _pre_atoms_body(x_ref, g_ref, b_ref, w_ref, o_ref):
    h = _ln(x_ref[...], g_ref[...], b_ref[...])
    o_ref[...] = jnp.dot(h, w_ref[...], preferred_element_type=jnp.float32)


def _pre_atoms(a_x, g, b, w, br=2000):
    n = a_x.shape[0]
    return pl.pallas_call(
        _pre_atoms_body,
        grid=(n // br,),
        in_specs=[pl.BlockSpec((br, HID), lambda i: (i, 0)),
                  _full((1, HID)), _full((1, HID)), _full((HID, HID))],
        out_specs=pl.BlockSpec((br, HID), lambda i: (i, 0)),
        out_shape=jax.ShapeDtypeStruct((n, HID), jnp.float32),
    )(a_x, g.reshape(1, HID), b.reshape(1, HID), w)


def _mesh_body(x_ref, g_ref, b_ref, wq_ref, bq_ref, wk_ref, bk_ref, wv_ref,
               bv_ref, wo_ref, bo_ref, w1_ref, mh_ref, hm_ref):
    x = _ln(x_ref[...], g_ref[...], b_ref[...])
    q = jnp.dot(x, wq_ref[...], preferred_element_type=jnp.float32) + bq_ref[...]
    k = jnp.dot(x, wk_ref[...], preferred_element_type=jnp.float32) + bk_ref[...]
    v = jnp.dot(x, wv_ref[...], preferred_element_type=jnp.float32) + bv_ref[...]
    dh = HID // NHEADS
    outs = []
    for h in range(NHEADS):
        qs = q[:, h * dh:(h + 1) * dh]
        ks = k[:, h * dh:(h + 1) * dh]
        vs = v[:, h * dh:(h + 1) * dh]
        att = lax.dot_general(qs, ks, (((1,), (1,)), ((), ())),
                              preferred_element_type=jnp.float32)
        att = jax.nn.softmax(att / math.sqrt(float(dh)), axis=-1)
        outs.append(jnp.dot(att, vs, preferred_element_type=jnp.float32))
    o = jnp.concatenate(outs, axis=1)
    mh = jnp.dot(o, wo_ref[...], preferred_element_type=jnp.float32) + bo_ref[...]
    mh_ref[...] = mh
    hm_ref[...] = jnp.dot(mh, w1_ref[...], preferred_element_type=jnp.float32)


def _mesh_branch(m_x, g, b, mha, w1):
    n = m_x.shape[0]
    vecs = [g, b, None, mha['bq'], None, mha['bk'], None, mha['bv'], None,
            mha['bo']]
    args = [m_x, g.reshape(1, HID), b.reshape(1, HID),
            mha['wq'], mha['bq'].reshape(1, HID),
            mha['wk'], mha['bk'].reshape(1, HID),
            mha['wv'], mha['bv'].reshape(1, HID),
            mha['wo'], mha['bo'].reshape(1, HID), w1]
    del vecs
    return pl.pallas_call(
        _mesh_body,
        out_shape=(jax.ShapeDtypeStruct((n, HID), jnp.float32),
                   jax.ShapeDtypeStruct((n, HID), jnp.float32)),
    )(*args)


def _filter_body(cutoff, ea_ref, ew_ref, h_ref, w1_ref, b1_ref, w2_ref,
                 b2_ref, o_ref):
    a1 = _ssp(jnp.dot(ea_ref[...], w1_ref[...],
                      preferred_element_type=jnp.float32) + b1_ref[...])
    w = jnp.dot(a1, w2_ref[...], preferred_element_type=jnp.float32) + b2_ref[...]
    ew = ew_ref[...]
    env = 0.5 * (jnp.cos(ew * (math.pi / cutoff)) + 1.0)
    env = jnp.where(ew <= cutoff, env, 0.0)
    o_ref[...] = w * env * h_ref[...]


def _edge_filter(ea, ew, hsrc, p, cutoff, be=1000):
    e = ea.shape[0]
    nrbf = ea.shape[1]
    body = functools.partial(_filter_body, cutoff)
    return pl.pallas_call(
        body,
        grid=(e // be,),
        in_specs=[pl.BlockSpec((be, nrbf), lambda i: (i, 0)),
                  pl.BlockSpec((be, 1), lambda i: (i, 0)),
                  pl.BlockSpec((be, HID), lambda i: (i, 0)),
                  _full((nrbf, HID)), _full((1, HID)),
                  _full((HID, HID)), _full((1, HID))],
        out_specs=pl.BlockSpec((be, HID), lambda i: (i, 0)),
        out_shape=jax.ShapeDtypeStruct((e, HID), jnp.float32),
    )(ea, ew.reshape(-1, 1), hsrc, p['mlp_w1'], p['mlp_b1'].reshape(1, HID),
      p['mlp_w2'], p['mlp_b2'].reshape(1, HID))


def _post_body(next_w, parts_ref, w2_ref, b2_ref, wl_ref, bl_ref, *rest):
    agg = parts_ref[0] + parts_ref[1]
    y = _ssp(jnp.dot(agg, w2_ref[...], preferred_element_type=jnp.float32)
             + b2_ref[...])
    y = jnp.dot(y, wl_ref[...], preferred_element_type=jnp.float32) + bl_ref[...]
    if next_w:
        nw_ref, y_ref, h2_ref = rest
        y_ref[...] = y
        h2_ref[...] = jnp.dot(y, nw_ref[...], preferred_element_type=jnp.float32)
    else:
        g_ref, b_ref, base_ref, res_ref, y_ref = rest
        y_ref[...] = base_ref[...] + _ln(y, g_ref[...], b_ref[...]) + res_ref[...]


def _post_interaction(parts, p, next_w, br):
    n = parts.shape[1]
    body = functools.partial(_post_body, True)
    return pl.pallas_call(
        body,
        grid=(n // br,),
        in_specs=[pl.BlockSpec((2, br, HID), lambda i: (0, i, 0)),
                  _full((HID, HID)), _full((1, HID)),
                  _full((HID, HID)), _full((1, HID)), _full((HID, HID))],
        out_specs=(pl.BlockSpec((br, HID), lambda i: (i, 0)),
                   pl.BlockSpec((br, HID), lambda i: (i, 0))),
        out_shape=(jax.ShapeDtypeStruct((n, HID), jnp.float32),
                   jax.ShapeDtypeStruct((n, HID), jnp.float32)),
    )(parts, p['lin2_w'], p['lin2_b'].reshape(1, HID),
      p['lin_w'], p['lin_b'].reshape(1, HID), next_w)


def _post_final(parts, p, g, b, base, res, br):
    n = parts.shape[1]
    body = functools.partial(_post_body, False)
    return pl.pallas_call(
        body,
        grid=(n // br,),
        in_specs=[pl.BlockSpec((2, br, HID), lambda i: (0, i, 0)),
                  _full((HID, HID)), _full((1, HID)),
                  _full((HID, HID)), _full((1, HID)),
                  _full((1, HID)), _full((1, HID)),
                  pl.BlockSpec((br, HID), lambda i: (i, 0)),
                  pl.BlockSpec((br, HID), lambda i: (i, 0))],
        out_specs=pl.BlockSpec((br, HID), lambda i: (i, 0)),
        out_shape=jax.ShapeDtypeStruct((n, HID), jnp.float32),
    )(parts, p['lin2_w'], p['lin2_b'].reshape(1, HID),
      p['lin_w'], p['lin_b'].reshape(1, HID),
      g.reshape(1, HID), b.reshape(1, HID), base, res)


def _make_sc_gather(e, n_rows, hid):
    assert e % KCH == 0
    nchunk = e // KCH
    mesh = plsc.VectorSubcoreMesh(core_axis_name="c", subcore_axis_name="s",
                                  num_cores=NC, num_subcores=NS)

    if nchunk % 2 == 0:
        npair = nchunk // 2

        def body(table_hbm, idx_hbm, out_hbm, idx0_v, idx1_v, rows0_v,
                 rows1_v, sem0, sem1):
            wid = lax.axis_index("s") * NC + lax.axis_index("c")
            my_n = (npair - 1 - wid) // NW + 1

            def step(i, carry):
                base0 = (wid + i * NW) * 2 * KCH
                base1 = base0 + KCH
                pltpu.sync_copy(idx_hbm.at[pl.ds(base0, KCH)], idx0_v)
                h0 = pltpu.async_copy(table_hbm.at[idx0_v], rows0_v, sem0)
                pltpu.sync_copy(idx_hbm.at[pl.ds(base1, KCH)], idx1_v)
                h1 = pltpu.async_copy(table_hbm.at[idx1_v], rows1_v, sem1)
                h0.wait()
                pltpu.sync_copy(rows0_v, out_hbm.at[pl.ds(base0, KCH)])
                h1.wait()
                pltpu.sync_copy(rows1_v, out_hbm.at[pl.ds(base1, KCH)])
                return carry

            lax.fori_loop(0, my_n, step, 0)

        return pl.kernel(
            body,
            out_type=jax.ShapeDtypeStruct((e, hid), jnp.float32),
            mesh=mesh,
            scratch_types=[pltpu.VMEM((KCH,), jnp.int32),
                           pltpu.VMEM((KCH,), jnp.int32),
                           pltpu.VMEM((KCH, hid), jnp.float32),
                           pltpu.VMEM((KCH, hid), jnp.float32),
                           pltpu.SemaphoreType.DMA,
                           pltpu.SemaphoreType.DMA],
        )

    def body(table_hbm, idx_hbm, out_hbm, idx_v, rows_v, sem):
        wid = lax.axis_index("s") * NC + lax.axis_index("c")
        my_n = (nchunk - 1 - wid) // NW + 1

        def step(i, carry):
            base = (wid + i * NW) * KCH
            pltpu.sync_copy(idx_hbm.at[pl.ds(base, KCH)], idx_v)
            pltpu.async_copy(table_hbm.at[idx_v], rows_v, sem).wait()
            pltpu.sync_copy(rows_v, out_hbm.at[pl.ds(base, KCH)])
            return carry

        lax.fori_loop(0, my_n, step, 0)

    return pl.kernel(
        body,
        out_type=jax.ShapeDtypeStruct((e, hid), jnp.float32),
        mesh=mesh,
        scratch_types=[pltpu.VMEM((KCH,), jnp.int32),
                       pltpu.VMEM((KCH, hid), jnp.float32),
                       pltpu.SemaphoreType.DMA],
    )


def _make_sc_scatter(e, n_rows, hid):
    assert e % KCH == 0 and n_rows % 8 == 0
    nchunk = e // KCH
    grp = n_rows // 8
    g_per, g_rem = grp // NS, grp % NS
    big, small = (g_per + 1) * 8, g_per * 8
    mesh = plsc.VectorSubcoreMesh(core_axis_name="c", subcore_axis_name="s",
                                  num_cores=NC, num_subcores=NS)

    del big, small

    paired = nchunk % 2 == 0
    npair = nchunk // 2

    def body(msg_hbm, dst_hbm, out_hbm, acc_sh, idx_v, msg_v, piece_v, sem,
             *extra):
        c = lax.axis_index("c")
        s = lax.axis_index("s")
        wid = s * NC + c
        r0 = (s * g_per + jnp.minimum(s, g_rem)) * 8
        ng = g_per + jnp.where(s < g_rem, 1, 0)

        zero = jnp.zeros((16,), jnp.float32)

        def zrow(r, carry):
            for j in range(hid // 16):
                piece_v[r, pl.ds(j * 16, 16)] = zero
            return carry

        lax.fori_loop(0, 8, zrow, 0)

        def zinit(q, carry):
            pltpu.sync_copy(piece_v, acc_sh.at[pl.ds(r0 + q * 8, 8)])
            return carry

        lax.fori_loop(0, ng, zinit, 0)
        plsc.subcore_barrier()

        if paired:
            idx1_v, msg1_v, sem1 = extra

            def step(i, carry):
                base0 = (wid + i * NW) * 2 * KCH
                base1 = base0 + KCH
                pltpu.sync_copy(dst_hbm.at[pl.ds(base0, KCH)], idx_v)
                h0 = pltpu.async_copy(msg_hbm.at[pl.ds(base0, KCH)], msg_v,
                                      sem)
                pltpu.sync_copy(dst_hbm.at[pl.ds(base1, KCH)], idx1_v)
                h1 = pltpu.async_copy(msg_hbm.at[pl.ds(base1, KCH)], msg1_v,
                                      sem1)
                h0.wait()
                pltpu.sync_copy(msg_v, acc_sh.at[idx_v], add=True)
                h1.wait()
                pltpu.sync_copy(msg1_v, acc_sh.at[idx1_v], add=True)
                return carry

            lax.fori_loop(0, (npair - 1 - wid) // NW + 1, step, 0)
        else:
            def step(i, carry):
                base = (wid + i * NW) * KCH
                pltpu.sync_copy(dst_hbm.at[pl.ds(base, KCH)], idx_v)
                pltpu.sync_copy(msg_hbm.at[pl.ds(base, KCH)], msg_v)
                pltpu.sync_copy(msg_v, acc_sh.at[idx_v], add=True)
                return carry

            lax.fori_loop(0, (nchunk - 1 - wid) // NW + 1, step, 0)
        plsc.subcore_barrier()

        def copy_out(q, carry):
            pltpu.sync_copy(acc_sh.at[pl.ds(r0 + q * 8, 8)], piece_v)
            pltpu.sync_copy(piece_v, out_hbm.at[c, pl.ds(r0 + q * 8, 8)])
            return carry

        lax.fori_loop(0, ng, copy_out, 0)

    scratch = [pltpu.VMEM_SHARED((n_rows, hid), jnp.float32),
               pltpu.VMEM((KCH,), jnp.int32),
               pltpu.VMEM((KCH, hid), jnp.float32),
               pltpu.VMEM((8, hid), jnp.float32),
               pltpu.SemaphoreType.DMA]
    if paired:
        scratch += [pltpu.VMEM((KCH,), jnp.int32),
                    pltpu.VMEM((KCH, hid), jnp.float32),
                    pltpu.SemaphoreType.DMA]

    return pl.kernel(
        body,
        out_type=jax.ShapeDtypeStruct((NC, n_rows, hid), jnp.float32),
        mesh=mesh,
        scratch_types=scratch,
    )


def kernel(a_x, m_x, a2a_edge_index, a2m_edge_index, m2a_edge_index,
           a2a_edge_weights, a2m_edge_weights, m2a_edge_weights,
           a2a_edge_attr, a2m_edge_attr, m2a_edge_attr, params):
    p = params
    n_atom = a_x.shape[0]
    n_mesh = m_x.shape[0]
    e_aa = a2a_edge_index.shape[1]
    e_am = a2m_edge_index.shape[1]

    src_aa, dst_aa = a2a_edge_index[0], a2a_edge_index[1]
    src_a2m, dst_a2m = a2m_edge_index[0], a2m_edge_index[1]
    src_m2a, dst_m2a = m2a_edge_index[0], m2a_edge_index[1]

    h_aa = _pre_atoms(a_x, p['ln_short_g'], p['ln_short_b'], p['short']['lin1_w'])
    m_h, h_m2a = _mesh_branch(m_x, p['ln_long_g'], p['ln_long_b'], p['mha'],
                              p['m2a']['lin1_w'])

    hsrc_aa = _make_sc_gather(e_aa, n_atom, HID)(h_aa, src_aa)
    hsrc_m2a = _make_sc_gather(e_am, n_mesh, HID)(h_m2a, src_m2a)

    msg_aa = _edge_filter(a2a_edge_attr, a2a_edge_weights, hsrc_aa,
                          p['short'], ATOM_CUTOFF)
    parts_aa = _make_sc_scatter(e_aa, n_atom, HID)(msg_aa, dst_aa)
    a_h, h_a2m = _post_interaction(parts_aa, p['short'], p['a2m']['lin1_w'],
                                   br=2000)

    msg_m2a = _edge_filter(m2a_edge_attr, m2a_edge_weights, hsrc_m2a,
                           p['m2a'], GRID_CUTOFF)
    parts_m2a = _make_sc_scatter(e_am, n_atom, HID)(msg_m2a, dst_m2a)

    hsrc_a2m = _make_sc_gather(e_am, n_atom, HID)(h_a2m, src_a2m)
    msg_a2m = _edge_filter(a2m_edge_attr, a2m_edge_weights, hsrc_a2m,
                           p['a2m'], GRID_CUTOFF)
    parts_a2m = _make_sc_scatter(e_am, n_mesh, HID)(msg_a2m, dst_a2m)

    out_a = _post_final(parts_m2a, p['m2a'], p['ln_m2a_g'], p['ln_m2a_b'],
                        a_h, a_x, br=2000)
    out_m = _post_final(parts_a2m, p['a2m'], p['ln_a2m_g'], p['ln_a2m_b'],
                        m_h, m_x, br=512)
    return out_a, out_m


# speedup vs baseline: 1.0510x; 1.0510x over previous
import functools
import math

import jax
import jax.numpy as jnp
from jax import lax
from jax.experimental import pallas as pl
from jax.experimental.pallas import tpu as pltpu
from jax.experimental.pallas import tpu_sc as plsc

HID = 128
NHEADS = 8
ATOM_CUTOFF = 5.0
GRID_CUTOFF = 8.0
LOG2 = math.log(2.0)

NC = 2
NS = 16
NW = NC * NS
KCH = 128


def _ssp(x):
    return jax.nn.softplus(x) - LOG2


def _ln(x, g, b):
    m = jnp.mean(x, axis=-1, keepdims=True)
    v = jnp.var(x, axis=-1, keepdims=True)
    return (x - m) * jax.lax.rsqrt(v + 1e-5) * g + b


def _full(shape):
    return pl.BlockSpec(shape, lambda *_: tuple(0 for _ in shape))


def _pre_atoms_body(x_ref, g_ref, b_ref, w_ref, o_ref):
    h = _ln(x_ref[...], g_ref[...], b_ref[...])
    o_ref[...] = jnp.dot(h, w_ref[...], preferred_element_type=jnp.float32)


def _pre_atoms(a_x, g, b, w, br=2000):
    n = a_x.shape[0]
    return pl.pallas_call(
        _pre_atoms_body,
        grid=(n // br,),
        in_specs=[pl.BlockSpec((br, HID), lambda i: (i, 0)),
                  _full((1, HID)), _full((1, HID)), _full((HID, HID))],
        out_specs=pl.BlockSpec((br, HID), lambda i: (i, 0)),
        out_shape=jax.ShapeDtypeStruct((n, HID), jnp.float32),
    )(a_x, g.reshape(1, HID), b.reshape(1, HID), w)


def _mesh_body(x_ref, g_ref, b_ref, wq_ref, bq_ref, wk_ref, bk_ref, wv_ref,
               bv_ref, wo_ref, bo_ref, w1_ref, mh_ref, hm_ref):
    x = _ln(x_ref[...], g_ref[...], b_ref[...])
    q = jnp.dot(x, wq_ref[...], preferred_element_type=jnp.float32) + bq_ref[...]
    k = jnp.dot(x, wk_ref[...], preferred_element_type=jnp.float32) + bk_ref[...]
    v = jnp.dot(x, wv_ref[...], preferred_element_type=jnp.float32) + bv_ref[...]
    dh = HID // NHEADS
    outs = []
    for h in range(NHEADS):
        qs = q[:, h * dh:(h + 1) * dh]
        ks = k[:, h * dh:(h + 1) * dh]
        vs = v[:, h * dh:(h + 1) * dh]
        att = lax.dot_general(qs, ks, (((1,), (1,)), ((), ())),
                              preferred_element_type=jnp.float32)
        att = jax.nn.softmax(att / math.sqrt(float(dh)), axis=-1)
        outs.append(jnp.dot(att, vs, preferred_element_type=jnp.float32))
    o = jnp.concatenate(outs, axis=1)
    mh = jnp.dot(o, wo_ref[...], preferred_element_type=jnp.float32) + bo_ref[...]
    mh_ref[...] = mh
    hm_ref[...] = jnp.dot(mh, w1_ref[...], preferred_element_type=jnp.float32)


def _mesh_branch(m_x, g, b, mha, w1):
    n = m_x.shape[0]
    vecs = [g, b, None, mha['bq'], None, mha['bk'], None, mha['bv'], None,
            mha['bo']]
    args = [m_x, g.reshape(1, HID), b.reshape(1, HID),
            mha['wq'], mha['bq'].reshape(1, HID),
            mha['wk'], mha['bk'].reshape(1, HID),
            mha['wv'], mha['bv'].reshape(1, HID),
            mha['wo'], mha['bo'].reshape(1, HID), w1]
    del vecs
    return pl.pallas_call(
        _mesh_body,
        out_shape=(jax.ShapeDtypeStruct((n, HID), jnp.float32),
                   jax.ShapeDtypeStruct((n, HID), jnp.float32)),
    )(*args)


def _filter_body(cutoff, ea_ref, ew_ref, h_ref, w1_ref, b1_ref, w2_ref,
                 b2_ref, o_ref):
    a1 = _ssp(jnp.dot(ea_ref[...], w1_ref[...],
                      preferred_element_type=jnp.float32) + b1_ref[...])
    w = jnp.dot(a1, w2_ref[...], preferred_element_type=jnp.float32) + b2_ref[...]
    ew = ew_ref[...]
    env = 0.5 * (jnp.cos(ew * (math.pi / cutoff)) + 1.0)
    env = jnp.where(ew <= cutoff, env, 0.0)
    o_ref[...] = w * env * h_ref[...]


def _edge_filter(ea, ew, hsrc, p, cutoff, be=1000):
    e = ea.shape[0]
    nrbf = ea.shape[1]
    body = functools.partial(_filter_body, cutoff)
    return pl.pallas_call(
        body,
        grid=(e // be,),
        in_specs=[pl.BlockSpec((be, nrbf), lambda i: (i, 0)),
                  pl.BlockSpec((be, 1), lambda i: (i, 0)),
                  pl.BlockSpec((be, HID), lambda i: (i, 0)),
                  _full((nrbf, HID)), _full((1, HID)),
                  _full((HID, HID)), _full((1, HID))],
        out_specs=pl.BlockSpec((be, HID), lambda i: (i, 0)),
        out_shape=jax.ShapeDtypeStruct((e, HID), jnp.float32),
    )(ea, ew.reshape(-1, 1), hsrc, p['mlp_w1'], p['mlp_b1'].reshape(1, HID),
      p['mlp_w2'], p['mlp_b2'].reshape(1, HID))


def _post_body(next_w, parts_ref, w2_ref, b2_ref, wl_ref, bl_ref, *rest):
    agg = parts_ref[0] + parts_ref[1]
    y = _ssp(jnp.dot(agg, w2_ref[...], preferred_element_type=jnp.float32)
             + b2_ref[...])
    y = jnp.dot(y, wl_ref[...], preferred_element_type=jnp.float32) + bl_ref[...]
    if next_w:
        nw_ref, y_ref, h2_ref = rest
        y_ref[...] = y
        h2_ref[...] = jnp.dot(y, nw_ref[...], preferred_element_type=jnp.float32)
    else:
        g_ref, b_ref, base_ref, res_ref, y_ref = rest
        y_ref[...] = base_ref[...] + _ln(y, g_ref[...], b_ref[...]) + res_ref[...]


def _post_interaction(parts, p, next_w, br):
    n = parts.shape[1]
    body = functools.partial(_post_body, True)
    return pl.pallas_call(
        body,
        grid=(n // br,),
        in_specs=[pl.BlockSpec((2, br, HID), lambda i: (0, i, 0)),
                  _full((HID, HID)), _full((1, HID)),
                  _full((HID, HID)), _full((1, HID)), _full((HID, HID))],
        out_specs=(pl.BlockSpec((br, HID), lambda i: (i, 0)),
                   pl.BlockSpec((br, HID), lambda i: (i, 0))),
        out_shape=(jax.ShapeDtypeStruct((n, HID), jnp.float32),
                   jax.ShapeDtypeStruct((n, HID), jnp.float32)),
    )(parts, p['lin2_w'], p['lin2_b'].reshape(1, HID),
      p['lin_w'], p['lin_b'].reshape(1, HID), next_w)


def _post_final(parts, p, g, b, base, res, br):
    n = parts.shape[1]
    body = functools.partial(_post_body, False)
    return pl.pallas_call(
        body,
        grid=(n // br,),
        in_specs=[pl.BlockSpec((2, br, HID), lambda i: (0, i, 0)),
                  _full((HID, HID)), _full((1, HID)),
                  _full((HID, HID)), _full((1, HID)),
                  _full((1, HID)), _full((1, HID)),
                  pl.BlockSpec((br, HID), lambda i: (i, 0)),
                  pl.BlockSpec((br, HID), lambda i: (i, 0))],
        out_specs=pl.BlockSpec((br, HID), lambda i: (i, 0)),
        out_shape=jax.ShapeDtypeStruct((n, HID), jnp.float32),
    )(parts, p['lin2_w'], p['lin2_b'].reshape(1, HID),
      p['lin_w'], p['lin_b'].reshape(1, HID),
      g.reshape(1, HID), b.reshape(1, HID), base, res)


def _make_sc_gather(e, n_rows, hid):
    assert e % KCH == 0 and n_rows % 8 == 0
    nchunk = e // KCH
    grp = n_rows // 8
    g_per, g_rem = grp // NS, grp % NS
    mesh = plsc.VectorSubcoreMesh(core_axis_name="c", subcore_axis_name="s",
                                  num_cores=NC, num_subcores=NS)

    def body(table_hbm, idx_hbm, out_hbm, tab_sh, idx_v, rows_v, piece_v):
        c = lax.axis_index("c")
        s = lax.axis_index("s")
        wid = s * NC + c
        r0 = (s * g_per + jnp.minimum(s, g_rem)) * 8
        ng = g_per + jnp.where(s < g_rem, 1, 0)

        def load(q, carry):
            pltpu.sync_copy(table_hbm.at[pl.ds(r0 + q * 8, 8)], piece_v)
            pltpu.sync_copy(piece_v, tab_sh.at[pl.ds(r0 + q * 8, 8)])
            return carry

        lax.fori_loop(0, ng, load, 0)
        plsc.subcore_barrier()

        my_n = (nchunk - 1 - wid) // NW + 1

        def step(i, carry):
            base = (wid + i * NW) * KCH
            pltpu.sync_copy(idx_hbm.at[pl.ds(base, KCH)], idx_v)
            pltpu.sync_copy(tab_sh.at[idx_v], rows_v)
            pltpu.sync_copy(rows_v, out_hbm.at[pl.ds(base, KCH)])
            return carry

        lax.fori_loop(0, my_n, step, 0)

    return pl.kernel(
        body,
        out_type=jax.ShapeDtypeStruct((e, hid), jnp.float32),
        mesh=mesh,
        scratch_types=[pltpu.VMEM_SHARED((n_rows, hid), jnp.float32),
                       pltpu.VMEM((KCH,), jnp.int32),
                       pltpu.VMEM((KCH, hid), jnp.float32),
                       pltpu.VMEM((8, hid), jnp.float32)],
    )


def _make_sc_scatter(e, n_rows, hid):
    assert e % KCH == 0 and n_rows % 8 == 0
    nchunk = e // KCH
    grp = n_rows // 8
    g_per, g_rem = grp // NS, grp % NS
    big, small = (g_per + 1) * 8, g_per * 8
    mesh = plsc.VectorSubcoreMesh(core_axis_name="c", subcore_axis_name="s",
                                  num_cores=NC, num_subcores=NS)

    del big, small

    paired = nchunk % 2 == 0
    npair = nchunk // 2

    def body(msg_hbm, dst_hbm, out_hbm, acc_sh, idx_v, msg_v, piece_v, sem,
             *extra):
        c = lax.axis_index("c")
        s = lax.axis_index("s")
        wid = s * NC + c
        r0 = (s * g_per + jnp.minimum(s, g_rem)) * 8
        ng = g_per + jnp.where(s < g_rem, 1, 0)

        zero = jnp.zeros((16,), jnp.float32)

        def zrow(r, carry):
            for j in range(hid // 16):
                piece_v[r, pl.ds(j * 16, 16)] = zero
            return carry

        lax.fori_loop(0, 8, zrow, 0)

        def zinit(q, carry):
            pltpu.sync_copy(piece_v, acc_sh.at[pl.ds(r0 + q * 8, 8)])
            return carry

        lax.fori_loop(0, ng, zinit, 0)
        plsc.subcore_barrier()

        if paired:
            idx1_v, msg1_v, sem1 = extra

            def step(i, carry):
                base0 = (wid + i * NW) * 2 * KCH
                base1 = base0 + KCH
                pltpu.sync_copy(dst_hbm.at[pl.ds(base0, KCH)], idx_v)
                h0 = pltpu.async_copy(msg_hbm.at[pl.ds(base0, KCH)], msg_v,
                                      sem)
                pltpu.sync_copy(dst_hbm.at[pl.ds(base1, KCH)], idx1_v)
                h1 = pltpu.async_copy(msg_hbm.at[pl.ds(base1, KCH)], msg1_v,
                                      sem1)
                h0.wait()
                pltpu.sync_copy(msg_v, acc_sh.at[idx_v], add=True)
                h1.wait()
                pltpu.sync_copy(msg1_v, acc_sh.at[idx1_v], add=True)
                return carry

            lax.fori_loop(0, (npair - 1 - wid) // NW + 1, step, 0)
        else:
            def step(i, carry):
                base = (wid + i * NW) * KCH
                pltpu.sync_copy(dst_hbm.at[pl.ds(base, KCH)], idx_v)
                pltpu.sync_copy(msg_hbm.at[pl.ds(base, KCH)], msg_v)
                pltpu.sync_copy(msg_v, acc_sh.at[idx_v], add=True)
                return carry

            lax.fori_loop(0, (nchunk - 1 - wid) // NW + 1, step, 0)
        plsc.subcore_barrier()

        def copy_out(q, carry):
            pltpu.sync_copy(acc_sh.at[pl.ds(r0 + q * 8, 8)], piece_v)
            pltpu.sync_copy(piece_v, out_hbm.at[c, pl.ds(r0 + q * 8, 8)])
            return carry

        lax.fori_loop(0, ng, copy_out, 0)

    scratch = [pltpu.VMEM_SHARED((n_rows, hid), jnp.float32),
               pltpu.VMEM((KCH,), jnp.int32),
               pltpu.VMEM((KCH, hid), jnp.float32),
               pltpu.VMEM((8, hid), jnp.float32),
               pltpu.SemaphoreType.DMA]
    if paired:
        scratch += [pltpu.VMEM((KCH,), jnp.int32),
                    pltpu.VMEM((KCH, hid), jnp.float32),
                    pltpu.SemaphoreType.DMA]

    return pl.kernel(
        body,
        out_type=jax.ShapeDtypeStruct((NC, n_rows, hid), jnp.float32),
        mesh=mesh,
        scratch_types=scratch,
    )


def kernel(a_x, m_x, a2a_edge_index, a2m_edge_index, m2a_edge_index,
           a2a_edge_weights, a2m_edge_weights, m2a_edge_weights,
           a2a_edge_attr, a2m_edge_attr, m2a_edge_attr, params):
    p = params
    n_atom = a_x.shape[0]
    n_mesh = m_x.shape[0]
    e_aa = a2a_edge_index.shape[1]
    e_am = a2m_edge_index.shape[1]

    src_aa, dst_aa = a2a_edge_index[0], a2a_edge_index[1]
    src_a2m, dst_a2m = a2m_edge_index[0], a2m_edge_index[1]
    src_m2a, dst_m2a = m2a_edge_index[0], m2a_edge_index[1]

    h_aa = _pre_atoms(a_x, p['ln_short_g'], p['ln_short_b'], p['short']['lin1_w'])
    m_h, h_m2a = _mesh_branch(m_x, p['ln_long_g'], p['ln_long_b'], p['mha'],
                              p['m2a']['lin1_w'])

    hsrc_aa = _make_sc_gather(e_aa, n_atom, HID)(h_aa, src_aa)
    hsrc_m2a = _make_sc_gather(e_am, n_mesh, HID)(h_m2a, src_m2a)

    msg_aa = _edge_filter(a2a_edge_attr, a2a_edge_weights, hsrc_aa,
                          p['short'], ATOM_CUTOFF)
    parts_aa = _make_sc_scatter(e_aa, n_atom, HID)(msg_aa, dst_aa)
    a_h, h_a2m = _post_interaction(parts_aa, p['short'], p['a2m']['lin1_w'],
                                   br=2000)

    msg_m2a = _edge_filter(m2a_edge_attr, m2a_edge_weights, hsrc_m2a,
                           p['m2a'], GRID_CUTOFF)
    parts_m2a = _make_sc_scatter(e_am, n_atom, HID)(msg_m2a, dst_m2a)

    hsrc_a2m = _make_sc_gather(e_am, n_atom, HID)(h_a2m, src_a2m)
    msg_a2m = _edge_filter(a2m_edge_attr, a2m_edge_weights, hsrc_a2m,
                           p['a2m'], GRID_CUTOFF)
    parts_a2m = _make_sc_scatter(e_am, n_mesh, HID)(msg_a2m, dst_a2m)

    out_a = _post_final(parts_m2a, p['m2a'], p['ln_m2a_g'], p['ln_m2a_b'],
                        a_h, a_x, br=2000)
    out_m = _post_final(parts_a2m, p['a2m'], p['ln_a2m_g'], p['ln_a2m_b'],
                        m_h, m_x, br=512)
    return out_a, out_m
